# baseline (speedup 1.0000x reference)
import functools

import jax
import jax.numpy as jnp
from jax import lax
from jax.experimental import pallas as pl
from jax.experimental.pallas import tpu as pltpu
from jax.experimental.pallas import tpu_sc as plsc

_LANES = 16


@functools.lru_cache(maxsize=None)
def _make_scatter_kernel(M, D, B, group):
  mesh = plsc.VectorSubcoreMesh(core_axis_name="c", subcore_axis_name="s")
  nc, ns = mesh.num_cores, mesh.num_subcores
  nw = nc * ns
  cols_per_w = D // nw

  ub = -(-M // (3 * _LANES)) * _LANES
  unit_lo = (0, ub, 2 * ub)
  unit_sz = (ub, ub, M - 2 * ub)
  n_units = cols_per_w * 3
  n_groups = B // _LANES // group

  @functools.partial(
      pl.kernel,
      out_type=jax.ShapeDtypeStruct((D * M,), jnp.float32),
      mesh=mesh,
      scratch_types=[
          pltpu.VMEM((ub,), jnp.float32),
          pltpu.VMEM((ub,), jnp.float32),
          pltpu.VMEM((B,), jnp.int32),
          pltpu.VMEM((B,), jnp.float32),
          pltpu.SemaphoreType.DMA,
          pltpu.SemaphoreType.DMA,
          pltpu.SemaphoreType.DMA,
          pltpu.SemaphoreType.DMA,
          pltpu.SemaphoreType.DMA,
      ],
      compiler_params=pltpu.CompilerParams(needs_layout_passes=False),
  )
  def scatter_kernel(inpT, idxT, srcT, outT, buf0, buf1, idxcol, srccol,
                     sl0, sl1, ss0, ss1, sio):
    wid = lax.axis_index("s") * nc + lax.axis_index("c")
    bufs = (buf0, buf1)
    sls = (sl0, sl1)
    sss = (ss0, ss1)

    def col_j(c):
      return wid * cols_per_w + c

    def load_desc(g):
      c, u = divmod(g, 3)
      b = g % 2
      return pltpu.make_async_copy(
          inpT.at[pl.ds(col_j(c) * M + unit_lo[u], unit_sz[u])],
          bufs[b].at[pl.ds(0, unit_sz[u])], sls[b])

    def store_desc(g):
      c, u = divmod(g, 3)
      b = g % 2
      return pltpu.make_async_copy(
          bufs[b].at[pl.ds(0, unit_sz[u])],
          outT.at[pl.ds(col_j(c) * M + unit_lo[u], unit_sz[u])], sss[b])

    def io_descs(c):
      return (pltpu.make_async_copy(idxT.at[pl.ds(col_j(c) * B, B)],
                                    idxcol, sio),
              pltpu.make_async_copy(srcT.at[pl.ds(col_j(c) * B, B)],
                                    srccol, sio))

    def compute(g):
      u = g % 3
      b = g % 2
      ubuf = bufs[b]
      lov = jnp.int32(unit_lo[u])
      hiv = jnp.int32(unit_lo[u] + unit_sz[u])

      def group_body(t, carry):
        base = t * (group * _LANES)
        ent = []
        for k in range(group):
          off = base + k * _LANES
          idxv = idxcol[pl.ds(off, _LANES)]
          srcv = srccol[pl.ds(off, _LANES)]
          inb = (idxv >= lov) & (idxv < hiv)
          _, keep = plsc.scan_count(idxv, mask=inb)
          ent.append((idxv - lov, srcv, keep))
        for a, s, m in ent:
          plsc.store_scatter(ubuf, [a], s, mask=m)
        return carry

      lax.fori_loop(0, n_groups, group_body, 0)

    di, dsv = io_descs(0)
    di.start()
    dsv.start()
    load_desc(0).start()

    for g in range(n_units):
      if g + 1 < n_units:
        if g >= 1:
          store_desc(g - 1).wait()
        load_desc(g + 1).start()
      if g % 3 == 0:
        di, dsv = io_descs(g // 3)
        di.wait()
        dsv.wait()
      load_desc(g).wait()
      compute(g)
      if g % 3 == 2 and g // 3 + 1 < cols_per_w:
        di, dsv = io_descs(g // 3 + 1)
        di.start()
        dsv.start()
      store_desc(g).start()

    store_desc(n_units - 2).wait()
    store_desc(n_units - 1).wait()

  return scatter_kernel


def kernel(input, dim, index, src):
  M, D = input.shape
  B = index.shape[0]
  idx = index + jnp.asarray(dim, index.dtype)
  f = _make_scatter_kernel(M, D, B, 8)
  outT = f(input.T.reshape(-1), idx.T.reshape(-1), src.T.reshape(-1))
  return outT.reshape(D, M).T


# speedup vs baseline: 1.0066x; 1.0066x over previous
import functools

import jax
import jax.numpy as jnp
from jax import lax
from jax.experimental import pallas as pl
from jax.experimental.pallas import tpu as pltpu
from jax.experimental.pallas import tpu_sc as plsc

_LANES = 16


@functools.lru_cache(maxsize=None)
def _make_scatter_kernel(M, D, B, group):
  mesh = plsc.VectorSubcoreMesh(core_axis_name="c", subcore_axis_name="s")
  nc, ns = mesh.num_cores, mesh.num_subcores
  nw = nc * ns
  cols_per_w = D // nw

  ub = -(-M // (3 * _LANES)) * _LANES
  unit_lo = (0, ub, 2 * ub)
  unit_sz = (ub, ub, M - 2 * ub)
  n_units = cols_per_w * 3
  n_groups = B // _LANES // group

  @functools.partial(
      pl.kernel,
      out_type=jax.ShapeDtypeStruct((D, M), jnp.float32),
      mesh=mesh,
      scratch_types=[
          pltpu.VMEM((ub,), jnp.float32),
          pltpu.VMEM((ub,), jnp.float32),
          pltpu.VMEM((B,), jnp.int32),
          pltpu.VMEM((B,), jnp.float32),
          pltpu.SemaphoreType.DMA,
          pltpu.SemaphoreType.DMA,
          pltpu.SemaphoreType.DMA,
          pltpu.SemaphoreType.DMA,
          pltpu.SemaphoreType.DMA,
      ],
      compiler_params=pltpu.CompilerParams(
          needs_layout_passes=False, use_tc_tiling_on_sc=False),
  )
  def scatter_kernel(inpT, idxT, srcT, outT, buf0, buf1, idxcol, srccol,
                     sl0, sl1, ss0, ss1, sio):
    wid = lax.axis_index("s") * nc + lax.axis_index("c")
    bufs = (buf0, buf1)
    sls = (sl0, sl1)
    sss = (ss0, ss1)

    def col_j(c):
      return wid * cols_per_w + c

    def load_desc(g):
      c, u = divmod(g, 3)
      b = g % 2
      return pltpu.make_async_copy(
          inpT.at[col_j(c), pl.ds(unit_lo[u], unit_sz[u])],
          bufs[b].at[pl.ds(0, unit_sz[u])], sls[b])

    def store_desc(g):
      c, u = divmod(g, 3)
      b = g % 2
      return pltpu.make_async_copy(
          bufs[b].at[pl.ds(0, unit_sz[u])],
          outT.at[col_j(c), pl.ds(unit_lo[u], unit_sz[u])], sss[b])

    def io_descs(c):
      return (pltpu.make_async_copy(idxT.at[col_j(c)], idxcol, sio),
              pltpu.make_async_copy(srcT.at[col_j(c)], srccol, sio))

    def compute(g):
      u = g % 3
      b = g % 2
      ubuf = bufs[b]
      lov = jnp.int32(unit_lo[u])
      hiv = jnp.int32(unit_lo[u] + unit_sz[u])

      def group_body(t, carry):
        base = t * (group * _LANES)
        ent = []
        for k in range(group):
          off = base + k * _LANES
          idxv = idxcol[pl.ds(off, _LANES)]
          srcv = srccol[pl.ds(off, _LANES)]
          inb = (idxv >= lov) & (idxv < hiv)
          _, keep = plsc.scan_count(idxv, mask=inb)
          ent.append((idxv - lov, srcv, keep))
        for a, s, m in ent:
          plsc.store_scatter(ubuf, [a], s, mask=m)
        return carry

      lax.fori_loop(0, n_groups, group_body, 0)

    di, dsv = io_descs(0)
    di.start()
    dsv.start()
    load_desc(0).start()

    for g in range(n_units):
      if g + 1 < n_units:
        if g >= 1:
          store_desc(g - 1).wait()
        load_desc(g + 1).start()
      if g % 3 == 0:
        di, dsv = io_descs(g // 3)
        di.wait()
        dsv.wait()
      load_desc(g).wait()
      compute(g)
      if g % 3 == 2 and g // 3 + 1 < cols_per_w:
        di, dsv = io_descs(g // 3 + 1)
        di.start()
        dsv.start()
      store_desc(g).start()

    store_desc(n_units - 2).wait()
    store_desc(n_units - 1).wait()

  return scatter_kernel


def kernel(input, dim, index, src):
  M, D = input.shape
  B = index.shape[0]
  idx = index + jnp.asarray(dim, index.dtype)
  f = _make_scatter_kernel(M, D, B, 8)
  outT = f(input.T, idx.T, src.T)
  return outT.T


# speedup vs baseline: 1.2360x; 1.2278x over previous
import functools

import jax
import jax.numpy as jnp
from jax import lax
from jax.experimental import pallas as pl
from jax.experimental.pallas import tpu as pltpu
from jax.experimental.pallas import tpu_sc as plsc

_LANES = 16
_SPLIT = 4


@functools.lru_cache(maxsize=None)
def _make_scatter_kernel(M, D, B, group):
  mesh = plsc.VectorSubcoreMesh(core_axis_name="c", subcore_axis_name="s")
  nc, ns = mesh.num_cores, mesh.num_subcores
  nw = nc * ns
  cols_per_w = D // nw
  mq = M // _SPLIT
  n_units = cols_per_w * _SPLIT
  n_groups = B // _LANES // group

  @functools.partial(
      pl.kernel,
      out_type=jax.ShapeDtypeStruct((D * _SPLIT, mq), jnp.float32),
      mesh=mesh,
      scratch_types=[
          pltpu.VMEM((mq,), jnp.float32),
          pltpu.VMEM((mq,), jnp.float32),
          pltpu.VMEM((B,), jnp.int32),
          pltpu.VMEM((B,), jnp.int32),
          pltpu.VMEM((B,), jnp.float32),
          pltpu.VMEM((B,), jnp.float32),
          pltpu.SemaphoreType.DMA,
          pltpu.SemaphoreType.DMA,
          pltpu.SemaphoreType.DMA,
          pltpu.SemaphoreType.DMA,
          pltpu.SemaphoreType.DMA,
          pltpu.SemaphoreType.DMA,
      ],
      compiler_params=pltpu.CompilerParams(needs_layout_passes=False),
  )
  def scatter_kernel(inpT, idxT, srcT, outT, buf0, buf1, idxc0, idxc1,
                     srcc0, srcc1, sl0, sl1, ss0, ss1, sio0, sio1):
    wid = lax.axis_index("s") * nc + lax.axis_index("c")
    bufs = (buf0, buf1)
    idxcs = (idxc0, idxc1)
    srccs = (srcc0, srcc1)
    sls = (sl0, sl1)
    sss = (ss0, ss1)
    sios = (sio0, sio1)

    def load_desc(g):
      b = g % 2
      return pltpu.make_async_copy(
          inpT.at[wid * n_units + g], bufs[b], sls[b])

    def store_desc(g):
      b = g % 2
      return pltpu.make_async_copy(
          bufs[b], outT.at[wid * n_units + g], sss[b])

    def io_descs(c):
      i = c % 2
      j = wid * cols_per_w + c
      return (pltpu.make_async_copy(idxT.at[j], idxcs[i], sios[i]),
              pltpu.make_async_copy(srcT.at[j], srccs[i], sios[i]))

    def compute(g):
      c, q = divmod(g, _SPLIT)
      b = g % 2
      ubuf = bufs[b]
      idxcol = idxcs[c % 2]
      srccol = srccs[c % 2]
      lov = jnp.int32(q * mq)
      hiv = jnp.int32((q + 1) * mq)

      def group_body(t, carry):
        base = t * (group * _LANES)
        ent = []
        for k in range(group):
          off = base + k * _LANES
          idxv = idxcol[pl.ds(off, _LANES)]
          srcv = srccol[pl.ds(off, _LANES)]
          inb = (idxv >= lov) & (idxv < hiv)
          _, keep = plsc.scan_count(idxv, mask=inb)
          ent.append((idxv - lov, srcv, keep))
        for a, s, m in ent:
          plsc.store_scatter(ubuf, [a], s, mask=m)
        return carry

      lax.fori_loop(0, n_groups, group_body, 0)

    di, dsv = io_descs(0)
    di.start()
    dsv.start()
    load_desc(0).start()

    for g in range(n_units):
      if g + 1 < n_units:
        if g >= 1:
          store_desc(g - 1).wait()
        load_desc(g + 1).start()
      if g % _SPLIT == 0:
        c = g // _SPLIT
        di, dsv = io_descs(c)
        di.wait()
        dsv.wait()
        if c + 1 < cols_per_w:
          di, dsv = io_descs(c + 1)
          di.start()
          dsv.start()
      load_desc(g).wait()
      compute(g)
      store_desc(g).start()

    store_desc(n_units - 2).wait()
    store_desc(n_units - 1).wait()

  return scatter_kernel


def kernel(input, dim, index, src):
  M, D = input.shape
  B = index.shape[0]
  idx = index + jnp.asarray(dim, index.dtype)
  f = _make_scatter_kernel(M, D, B, 8)
  outT = f(input.T.reshape(D * _SPLIT, M // _SPLIT), idx.T, src.T)
  return outT.reshape(D, M).T
